# Initial kernel scaffold; baseline (speedup 1.0000x reference)
import jax
import jax.numpy as jnp
from jax.experimental import pallas as pl


def kernel(data, remain_idx, global_token):
    raise NotImplementedError("write your pallas kernel here")


# speedup vs baseline: 2.5693x; 2.5693x over previous
import functools

import jax
import jax.numpy as jnp
from jax import lax
from jax.experimental import pallas as pl
from jax.experimental.pallas import tpu as pltpu
from jax.experimental.pallas import tpu_sc as plsc

_LANES = 16


def _positional_encoding(d_model, seq_len=1000):
    position = jnp.arange(seq_len, dtype=jnp.float32).reshape(-1, 1)
    i = jnp.arange(d_model) // 2
    exp_term = 2.0 * i.astype(jnp.float32) / d_model
    div_term = jnp.power(10000.0, exp_term).reshape(1, -1)
    pe = position / div_term
    pe = pe.at[:, 0::2].set(jnp.sin(pe[:, 0::2]))
    pe = pe.at[:, 1::2].set(jnp.cos(pe[:, 1::2]))
    return pe


@functools.partial(jax.jit, static_argnames=("B", "S", "R", "D"))
def _run(data_flat, idx_flat, pos, global_token, *, B, S, R, D):
    info = plsc.get_sparse_core_info()
    NC, NS = info.num_cores, info.num_subcores
    NW = NC * NS
    N = B * R
    assert N % NW == 0
    RW = N // NW
    assert R % RW == 0
    CH = 32
    assert RW % CH == 0 and CH % _LANES == 0 and D % _LANES == 0
    assert B == _LANES

    mesh = plsc.VectorSubcoreMesh(core_axis_name="c", subcore_axis_name="s")
    rep = jnp.broadcast_to(
        jnp.arange(S, dtype=jnp.int32)[:, None], (S, D))

    def body(data_hbm, idx_hbm, pos_hbm, gt_hbm, rep_hbm, out_hbm, ridx_hbm,
             idx_v, didx_v, pidx_v, oidx_v, ridxc_v, gidx_v,
             rows_v, pos_v, rbuf_v, ga_v, gb_v,
             sem_a, sem_b, sem_c):
        cid = lax.axis_index("c")
        sid = lax.axis_index("s")
        wid = sid * NC + cid
        base = wid * RW
        b = base // R
        out_base = b * (R + 1) + 1 + (base - b * R)

        pltpu.sync_copy(idx_hbm.at[pl.ds(base, RW)], idx_v)

        @pl.when(wid == 0)
        def _():
            pltpu.sync_copy(pos_hbm.at[pl.ds(0, 1)], gb_v)
            pltpu.sync_copy(gt_hbm, ga_v.at[pl.ds(0, 1)])

            def gadd(j, carry):
                sl = pl.ds(j * _LANES, _LANES)
                ga_v[0, sl] = ga_v[0, sl] + gb_v[0, sl]
                return carry

            lax.fori_loop(0, D // _LANES, gadd, 0)

            def gdup(r, carry):
                def jdup(j, cc):
                    sl = pl.ds(j * _LANES, _LANES)
                    ga_v[r, sl] = ga_v[0, sl]
                    return cc
                lax.fori_loop(0, D // _LANES, jdup, 0)
                return carry

            lax.fori_loop(1, B, gdup, 0)
            gidx_v[pl.ds(0, _LANES)] = (
                lax.iota(jnp.int32, _LANES) * (R + 1))
            pltpu.async_copy(ga_v, out_hbm.at[gidx_v], sem_a).wait()

        def chunk(ci, carry):
            off = ci * CH

            def ivec(i, c):
                sl = pl.ds(i * _LANES, _LANES)
                v = idx_v[pl.ds(off + i * _LANES, _LANES)]
                didx_v[sl] = v + b * S
                pidx_v[sl] = v + 1
                ridxc_v[sl] = v
                oidx_v[sl] = (lax.iota(jnp.int32, _LANES)
                              + (out_base + off + i * _LANES))
                return c

            lax.fori_loop(0, CH // _LANES, ivec, 0)

            cp_d = pltpu.async_copy(data_hbm.at[didx_v], rows_v, sem_a)
            cp_p = pltpu.async_copy(pos_hbm.at[pidx_v], pos_v, sem_b)
            cp_r = pltpu.async_copy(rep_hbm.at[ridxc_v], rbuf_v, sem_c)
            cp_d.wait()
            cp_p.wait()

            def radd(r, c):
                def jadd(j, cc):
                    sl = pl.ds(j * _LANES, _LANES)
                    rows_v[r, sl] = rows_v[r, sl] + pos_v[r, sl]
                    return cc

                lax.fori_loop(0, D // _LANES, jadd, 0)
                return c

            lax.fori_loop(0, CH, radd, 0)

            cp_r.wait()
            pltpu.async_copy(rows_v, out_hbm.at[oidx_v], sem_a).wait()
            pltpu.sync_copy(rbuf_v, ridx_hbm.at[pl.ds(base + off, CH)])
            return carry

        lax.fori_loop(0, RW // CH, chunk, 0)

    out_flat, ridx_flat = pl.kernel(
        body,
        out_type=(
            jax.ShapeDtypeStruct((B * (R + 1), D), jnp.float32),
            jax.ShapeDtypeStruct((N, D), jnp.int32),
        ),
        mesh=mesh,
        scratch_types=[
            pltpu.VMEM((RW,), jnp.int32),
            pltpu.VMEM((CH,), jnp.int32),
            pltpu.VMEM((CH,), jnp.int32),
            pltpu.VMEM((CH,), jnp.int32),
            pltpu.VMEM((CH,), jnp.int32),
            pltpu.VMEM((_LANES,), jnp.int32),
            pltpu.VMEM((CH, D), jnp.float32),
            pltpu.VMEM((CH, D), jnp.float32),
            pltpu.VMEM((CH, D), jnp.int32),
            pltpu.VMEM((B, D), jnp.float32),
            pltpu.VMEM((1, D), jnp.float32),
            pltpu.SemaphoreType.DMA,
            pltpu.SemaphoreType.DMA,
            pltpu.SemaphoreType.DMA,
        ],
    )(data_flat, idx_flat, pos, global_token, rep)
    return out_flat, ridx_flat


def kernel(data, remain_idx, global_token):
    B, S, D = data.shape
    R = remain_idx.shape[1]
    pos = _positional_encoding(D)[: S + 1, :]
    out_flat, ridx_flat = _run(
        data.reshape(B * S, D),
        remain_idx.reshape(B * R),
        pos,
        global_token,
        B=B, S=S, R=R, D=D,
    )
    return (out_flat.reshape(B, R + 1, D),
            ridx_flat.reshape(B, R, D))


# speedup vs baseline: 2.7075x; 1.0538x over previous
import functools

import jax
import jax.numpy as jnp
from jax import lax
from jax.experimental import pallas as pl
from jax.experimental.pallas import tpu as pltpu
from jax.experimental.pallas import tpu_sc as plsc

_LANES = 16


def _positional_encoding(d_model, seq_len=1000):
    position = jnp.arange(seq_len, dtype=jnp.float32).reshape(-1, 1)
    i = jnp.arange(d_model) // 2
    exp_term = 2.0 * i.astype(jnp.float32) / d_model
    div_term = jnp.power(10000.0, exp_term).reshape(1, -1)
    pe = position / div_term
    pe = pe.at[:, 0::2].set(jnp.sin(pe[:, 0::2]))
    pe = pe.at[:, 1::2].set(jnp.cos(pe[:, 1::2]))
    return pe


@functools.partial(jax.jit, static_argnames=("B", "S", "R", "D"))
def _run(data_flat, idx_flat, pos, global_token, *, B, S, R, D):
    info = plsc.get_sparse_core_info()
    NC, NS = info.num_cores, info.num_subcores
    NW = NC * NS
    N = B * R
    assert N % NW == 0
    RW = N // NW
    assert R % RW == 0
    CH = _LANES
    NCH = RW // CH
    assert NCH % 2 == 0 and D % _LANES == 0
    assert B == _LANES
    NV = D // _LANES

    mesh = plsc.VectorSubcoreMesh(core_axis_name="c", subcore_axis_name="s")

    def body(data_hbm, idx_hbm, pos_hbm, gt_hbm, out_hbm, ridx_hbm,
             idx_v,
             didx0, didx1, pidx0, pidx1, oidx0, oidx1, gidx_v,
             rows0, rows1, pos0, pos1, rbuf0, rbuf1,
             ga_v, gb_v,
             sd0, sd1, sp0, sp1, so0, so1, sr0, sr1, sg):
        didx = (didx0, didx1)
        pidx = (pidx0, pidx1)
        oidx = (oidx0, oidx1)
        rows = (rows0, rows1)
        posb = (pos0, pos1)
        rbuf = (rbuf0, rbuf1)
        sd = (sd0, sd1)
        sp = (sp0, sp1)
        so = (so0, so1)
        sr = (sr0, sr1)

        cid = lax.axis_index("c")
        sid = lax.axis_index("s")
        wid = sid * NC + cid
        base = wid * RW
        b = base // R
        out_base = b * (R + 1) + 1 + (base - b * R)

        pltpu.sync_copy(idx_hbm.at[pl.ds(base, RW)], idx_v)

        def set_indices(p, ci):
            off = ci * CH
            v = idx_v[pl.ds(off, CH)]
            didx[p][...] = v + b * S
            pidx[p][...] = v + 1
            oidx[p][...] = lax.iota(jnp.int32, _LANES) + (out_base + off)

        def issue_gathers(p):
            cp_d = pltpu.async_copy(data_hbm.at[didx[p]], rows[p], sd[p])
            cp_p = pltpu.async_copy(pos_hbm.at[pidx[p]], posb[p], sp[p])
            return cp_d, cp_p

        @pl.when(wid == 0)
        def _():
            pltpu.sync_copy(pos_hbm.at[pl.ds(0, 1)], gb_v)
            pltpu.sync_copy(gt_hbm, ga_v.at[pl.ds(0, 1)])

            def gadd(j, carry):
                sl = pl.ds(j * _LANES, _LANES)
                ga_v[0, sl] = ga_v[0, sl] + gb_v[0, sl]
                return carry

            lax.fori_loop(0, NV, gadd, 0, unroll=8)

            def gdup(r, carry):
                def jdup(j, cc):
                    sl = pl.ds(j * _LANES, _LANES)
                    ga_v[r, sl] = ga_v[0, sl]
                    return cc
                lax.fori_loop(0, NV, jdup, 0, unroll=8)
                return carry

            lax.fori_loop(1, B, gdup, 0)
            gidx_v[...] = lax.iota(jnp.int32, _LANES) * (R + 1)
            pltpu.async_copy(ga_v, out_hbm.at[gidx_v], sg).wait()

        set_indices(0, 0)
        issue_gathers(0)

        def process(p, ci):
            @pl.when(ci + 1 < NCH)
            def _():
                q = 1 - p
                @pl.when(ci >= 1)
                def _():
                    pltpu.make_async_copy(
                        rows[q], out_hbm.at[oidx[q]], so[q]).wait()
                    pltpu.make_async_copy(
                        rbuf[q], ridx_hbm.at[pl.ds(base, CH)], sr[q]).wait()
                set_indices(q, ci + 1)
                issue_gathers(q)

            pltpu.make_async_copy(data_hbm.at[didx[p]], rows[p], sd[p]).wait()
            pltpu.make_async_copy(pos_hbm.at[pidx[p]], posb[p], sp[p]).wait()

            def radd(r, c):
                def jadd(j, cc):
                    sl = pl.ds(j * _LANES, _LANES)
                    rows[p][r, sl] = rows[p][r, sl] + posb[p][r, sl]
                    return cc
                lax.fori_loop(0, NV, jadd, 0, unroll=8)
                return c
            lax.fori_loop(0, CH, radd, 0)

            iv = idx_v[pl.ds(ci * CH, CH)]

            for r in range(CH):
                val = jnp.broadcast_to(iv[r], (_LANES,))

                def jst(j, cc, r=r, val=val):
                    rbuf[p][r, pl.ds(j * _LANES, _LANES)] = val
                    return cc
                lax.fori_loop(0, NV, jst, 0, unroll=8)

            pltpu.async_copy(rows[p], out_hbm.at[oidx[p]], so[p])
            pltpu.async_copy(
                rbuf[p], ridx_hbm.at[pl.ds(base + ci * CH, CH)], sr[p])

        def pair(t, carry):
            process(0, 2 * t)
            process(1, 2 * t + 1)
            return carry

        lax.fori_loop(0, NCH // 2, pair, 0)

        pltpu.make_async_copy(
            rows[0], out_hbm.at[oidx[0]], so[0]).wait()
        pltpu.make_async_copy(
            rbuf[0], ridx_hbm.at[pl.ds(base, CH)], sr[0]).wait()
        pltpu.make_async_copy(
            rows[1], out_hbm.at[oidx[1]], so[1]).wait()
        pltpu.make_async_copy(
            rbuf[1], ridx_hbm.at[pl.ds(base, CH)], sr[1]).wait()

    out_flat, ridx_flat = pl.kernel(
        body,
        out_type=(
            jax.ShapeDtypeStruct((B * (R + 1), D), jnp.float32),
            jax.ShapeDtypeStruct((N, D), jnp.int32),
        ),
        mesh=mesh,
        scratch_types=[
            pltpu.VMEM((RW,), jnp.int32),
            pltpu.VMEM((CH,), jnp.int32),
            pltpu.VMEM((CH,), jnp.int32),
            pltpu.VMEM((CH,), jnp.int32),
            pltpu.VMEM((CH,), jnp.int32),
            pltpu.VMEM((CH,), jnp.int32),
            pltpu.VMEM((CH,), jnp.int32),
            pltpu.VMEM((_LANES,), jnp.int32),
            pltpu.VMEM((CH, D), jnp.float32),
            pltpu.VMEM((CH, D), jnp.float32),
            pltpu.VMEM((CH, D), jnp.float32),
            pltpu.VMEM((CH, D), jnp.float32),
            pltpu.VMEM((CH, D), jnp.int32),
            pltpu.VMEM((CH, D), jnp.int32),
            pltpu.VMEM((B, D), jnp.float32),
            pltpu.VMEM((1, D), jnp.float32),
            pltpu.SemaphoreType.DMA,
            pltpu.SemaphoreType.DMA,
            pltpu.SemaphoreType.DMA,
            pltpu.SemaphoreType.DMA,
            pltpu.SemaphoreType.DMA,
            pltpu.SemaphoreType.DMA,
            pltpu.SemaphoreType.DMA,
            pltpu.SemaphoreType.DMA,
            pltpu.SemaphoreType.DMA,
        ],
    )(data_flat, idx_flat, pos, global_token)
    return out_flat, ridx_flat


def kernel(data, remain_idx, global_token):
    B, S, D = data.shape
    R = remain_idx.shape[1]
    pos = _positional_encoding(D)[: S + 1, :]
    out_flat, ridx_flat = _run(
        data.reshape(B * S, D),
        remain_idx.reshape(B * R),
        pos,
        global_token,
        B=B, S=S, R=R, D=D,
    )
    return (out_flat.reshape(B, R + 1, D),
            ridx_flat.reshape(B, R, D))


# speedup vs baseline: 4.6593x; 1.7209x over previous
import functools

import jax
import jax.numpy as jnp
import numpy as np
from jax import lax
from jax.experimental import pallas as pl
from jax.experimental.pallas import tpu as pltpu
from jax.experimental.pallas import tpu_sc as plsc

_LANES = 16


def _positional_encoding_np(d_model, seq_len=1000):
    position = np.arange(seq_len, dtype=np.float32).reshape(-1, 1)
    i = np.arange(d_model) // 2
    exp_term = (2.0 * i.astype(np.float32) / d_model).astype(np.float32)
    div_term = np.power(np.float32(10000.0), exp_term).reshape(1, -1)
    pe = (position / div_term).astype(np.float32)
    pe[:, 0::2] = np.sin(pe[:, 0::2])
    pe[:, 1::2] = np.cos(pe[:, 1::2])
    return pe


@functools.partial(jax.jit, static_argnames=("B", "S", "R", "D"))
def _run(data_flat, idx_flat, pos, global_token, *, B, S, R, D):
    info = plsc.get_sparse_core_info()
    NC, NS = info.num_cores, info.num_subcores
    NW = NC * NS
    N = B * R
    assert N % NW == 0
    RW = N // NW
    assert R % RW == 0
    CH = _LANES
    NCH = RW // CH
    assert NCH % 2 == 0 and D % _LANES == 0
    assert B % 8 == 0
    NV = D // _LANES

    mesh = plsc.VectorSubcoreMesh(core_axis_name="c", subcore_axis_name="s")

    def body(data_hbm, idx_hbm, pos_hbm, gt_hbm, out_hbm, ridx_hbm,
             idx_v,
             didx0, didx1, pidx0, pidx1, oidx0, oidx1,
             rows0, rows1, pos0, pos1, rbuf0, rbuf1,
             ga_v, gb_v,
             sd0, sd1, sp0, sp1, so0, so1, sr0, sr1, sg):
        didx = (didx0, didx1)
        pidx = (pidx0, pidx1)
        oidx = (oidx0, oidx1)
        rows = (rows0, rows1)
        posb = (pos0, pos1)
        rbuf = (rbuf0, rbuf1)
        sd = (sd0, sd1)
        sp = (sp0, sp1)
        so = (so0, so1)
        sr = (sr0, sr1)

        cid = lax.axis_index("c")
        sid = lax.axis_index("s")
        wid = sid * NC + cid
        base = wid * RW
        b = base // R
        r0 = base - b * R

        pltpu.sync_copy(idx_hbm.at[pl.ds(base, RW)], idx_v)

        def set_indices(p, ci):
            off = ci * CH
            v = idx_v[pl.ds(off, CH)]
            didx[p][...] = v + b * S
            pidx[p][...] = v + 1
            oidx[p][...] = (lax.iota(jnp.int32, _LANES)
                            + (r0 + off + 1)) * B + b

        def issue_gathers(p):
            pltpu.async_copy(data_hbm.at[didx[p]], rows[p], sd[p])
            pltpu.async_copy(pos_hbm.at[pidx[p]], posb[p], sp[p])

        @pl.when(wid == 0)
        def _():
            pltpu.sync_copy(pos_hbm.at[pl.ds(0, 1)], gb_v)
            pltpu.sync_copy(gt_hbm, ga_v.at[pl.ds(0, 1)])

            def gadd(j, carry):
                sl = pl.ds(j * _LANES, _LANES)
                ga_v[0, sl] = ga_v[0, sl] + gb_v[0, sl]
                return carry

            lax.fori_loop(0, NV, gadd, 0, unroll=8)

            def gdup(r, carry):
                def jdup(j, cc):
                    sl = pl.ds(j * _LANES, _LANES)
                    ga_v[r, sl] = ga_v[0, sl]
                    return cc
                lax.fori_loop(0, NV, jdup, 0, unroll=8)
                return carry

            lax.fori_loop(1, B, gdup, 0)
            pltpu.async_copy(ga_v, out_hbm.at[pl.ds(0, B)], sg).wait()

        set_indices(0, 0)
        issue_gathers(0)

        def process(p, ci):
            @pl.when(ci + 1 < NCH)
            def _():
                q = 1 - p
                @pl.when(ci >= 1)
                def _():
                    pltpu.make_async_copy(
                        rows[q], out_hbm.at[oidx[q]], so[q]).wait()
                    pltpu.make_async_copy(
                        rbuf[q], ridx_hbm.at[pl.ds(base, CH)], sr[q]).wait()
                set_indices(q, ci + 1)
                issue_gathers(q)

            pltpu.make_async_copy(data_hbm.at[didx[p]], rows[p], sd[p]).wait()
            pltpu.make_async_copy(pos_hbm.at[pidx[p]], posb[p], sp[p]).wait()

            def radd(r, c):
                def jadd(j, cc):
                    sl = pl.ds(j * _LANES, _LANES)
                    rows[p][r, sl] = rows[p][r, sl] + posb[p][r, sl]
                    return cc
                lax.fori_loop(0, NV, jadd, 0, unroll=8)
                return c
            lax.fori_loop(0, CH, radd, 0)

            iv = idx_v[pl.ds(ci * CH, CH)]
            for r in range(CH):
                val = jnp.broadcast_to(iv[r], (_LANES,))

                def jst(j, cc, r=r, val=val):
                    rbuf[p][r, pl.ds(j * _LANES, _LANES)] = val
                    return cc
                lax.fori_loop(0, NV, jst, 0, unroll=8)

            pltpu.async_copy(rows[p], out_hbm.at[oidx[p]], so[p])
            pltpu.async_copy(
                rbuf[p], ridx_hbm.at[pl.ds(base + ci * CH, CH)], sr[p])

        def pair(t, carry):
            process(0, 2 * t)
            process(1, 2 * t + 1)
            return carry

        lax.fori_loop(0, NCH // 2, pair, 0)

        pltpu.make_async_copy(
            rows[0], out_hbm.at[oidx[0]], so[0]).wait()
        pltpu.make_async_copy(
            rbuf[0], ridx_hbm.at[pl.ds(base, CH)], sr[0]).wait()
        pltpu.make_async_copy(
            rows[1], out_hbm.at[oidx[1]], so[1]).wait()
        pltpu.make_async_copy(
            rbuf[1], ridx_hbm.at[pl.ds(base, CH)], sr[1]).wait()

    out_t_flat, ridx_flat = pl.kernel(
        body,
        out_type=(
            jax.ShapeDtypeStruct(((R + 1) * B, D), jnp.float32),
            jax.ShapeDtypeStruct((N, D), jnp.int32),
        ),
        mesh=mesh,
        scratch_types=[
            pltpu.VMEM((RW,), jnp.int32),
            pltpu.VMEM((CH,), jnp.int32),
            pltpu.VMEM((CH,), jnp.int32),
            pltpu.VMEM((CH,), jnp.int32),
            pltpu.VMEM((CH,), jnp.int32),
            pltpu.VMEM((CH,), jnp.int32),
            pltpu.VMEM((CH,), jnp.int32),
            pltpu.VMEM((CH, D), jnp.float32),
            pltpu.VMEM((CH, D), jnp.float32),
            pltpu.VMEM((CH, D), jnp.float32),
            pltpu.VMEM((CH, D), jnp.float32),
            pltpu.VMEM((CH, D), jnp.int32),
            pltpu.VMEM((CH, D), jnp.int32),
            pltpu.VMEM((B, D), jnp.float32),
            pltpu.VMEM((1, D), jnp.float32),
            pltpu.SemaphoreType.DMA,
            pltpu.SemaphoreType.DMA,
            pltpu.SemaphoreType.DMA,
            pltpu.SemaphoreType.DMA,
            pltpu.SemaphoreType.DMA,
            pltpu.SemaphoreType.DMA,
            pltpu.SemaphoreType.DMA,
            pltpu.SemaphoreType.DMA,
            pltpu.SemaphoreType.DMA,
        ],
    )(data_flat, idx_flat, pos, global_token)
    return out_t_flat, ridx_flat


def kernel(data, remain_idx, global_token):
    B, S, D = data.shape
    R = remain_idx.shape[1]
    pos = jnp.asarray(_positional_encoding_np(D)[: S + 1, :])
    out_t_flat, ridx_flat = _run(
        data.reshape(B * S, D),
        remain_idx.reshape(B * R),
        pos,
        global_token,
        B=B, S=S, R=R, D=D,
    )
    out = out_t_flat.reshape(R + 1, B, D).transpose(1, 0, 2)
    return (out, ridx_flat.reshape(B, R, D))


# speedup vs baseline: 5.4719x; 1.1744x over previous
import functools

import jax
import jax.numpy as jnp
import numpy as np
from jax import lax
from jax.experimental import pallas as pl
from jax.experimental.pallas import tpu as pltpu
from jax.experimental.pallas import tpu_sc as plsc

_LANES = 16


def _positional_encoding_np(d_model, seq_len=1000):
    position = np.arange(seq_len, dtype=np.float32).reshape(-1, 1)
    i = np.arange(d_model) // 2
    exp_term = (2.0 * i.astype(np.float32) / d_model).astype(np.float32)
    div_term = np.power(np.float32(10000.0), exp_term).reshape(1, -1)
    pe = (position / div_term).astype(np.float32)
    pe[:, 0::2] = np.sin(pe[:, 0::2])
    pe[:, 1::2] = np.cos(pe[:, 1::2])
    return pe


@functools.partial(jax.jit, static_argnames=("B", "S", "R", "D"))
def _run(data_flat, idx_flat, pos, global_token, *, B, S, R, D):
    info = plsc.get_sparse_core_info()
    NC, NS = info.num_cores, info.num_subcores
    NW = NC * NS
    N = B * R
    assert N % NW == 0
    RW = N // NW
    assert R % RW == 0
    CH = _LANES
    NCH = RW // CH
    assert NCH % 2 == 0 and D % _LANES == 0
    assert B % 8 == 0
    NV = D // _LANES

    mesh = plsc.VectorSubcoreMesh(core_axis_name="c", subcore_axis_name="s")

    def body(data_hbm, idx_hbm, pos_hbm, gt_hbm, out_hbm,
             idx_v,
             didx0, didx1, pidx0, pidx1, oidx0, oidx1,
             rows0, rows1, pos0, pos1,
             ga_v, gb_v,
             sd0, sd1, sp0, sp1, so0, so1, sg):
        didx = (didx0, didx1)
        pidx = (pidx0, pidx1)
        oidx = (oidx0, oidx1)
        rows = (rows0, rows1)
        posb = (pos0, pos1)
        sd = (sd0, sd1)
        sp = (sp0, sp1)
        so = (so0, so1)

        cid = lax.axis_index("c")
        sid = lax.axis_index("s")
        wid = sid * NC + cid
        base = wid * RW
        b = base // R
        r0 = base - b * R

        pltpu.sync_copy(idx_hbm.at[pl.ds(base, RW)], idx_v)

        def set_indices(p, ci):
            off = ci * CH
            v = idx_v[pl.ds(off, CH)]
            didx[p][...] = v + b * S
            pidx[p][...] = v + 1
            oidx[p][...] = (lax.iota(jnp.int32, _LANES)
                            + (r0 + off + 1)) * B + b

        def issue_gathers(p):
            pltpu.async_copy(data_hbm.at[didx[p]], rows[p], sd[p])
            pltpu.async_copy(pos_hbm.at[pidx[p]], posb[p], sp[p])

        @pl.when(wid == 0)
        def _():
            pltpu.sync_copy(pos_hbm.at[pl.ds(0, 1)], gb_v)
            pltpu.sync_copy(gt_hbm, ga_v.at[pl.ds(0, 1)])

            def gadd(j, carry):
                sl = pl.ds(j * _LANES, _LANES)
                ga_v[0, sl] = ga_v[0, sl] + gb_v[0, sl]
                return carry

            lax.fori_loop(0, NV, gadd, 0, unroll=8)

            def gdup(r, carry):
                def jdup(j, cc):
                    sl = pl.ds(j * _LANES, _LANES)
                    ga_v[r, sl] = ga_v[0, sl]
                    return cc
                lax.fori_loop(0, NV, jdup, 0, unroll=8)
                return carry

            lax.fori_loop(1, B, gdup, 0)
            pltpu.async_copy(ga_v, out_hbm.at[pl.ds(0, B)], sg).wait()

        set_indices(0, 0)
        issue_gathers(0)

        def process(p, ci):
            @pl.when(ci + 1 < NCH)
            def _():
                q = 1 - p
                @pl.when(ci >= 1)
                def _():
                    pltpu.make_async_copy(
                        rows[q], out_hbm.at[oidx[q]], so[q]).wait()
                set_indices(q, ci + 1)
                issue_gathers(q)

            pltpu.make_async_copy(data_hbm.at[didx[p]], rows[p], sd[p]).wait()
            pltpu.make_async_copy(pos_hbm.at[pidx[p]], posb[p], sp[p]).wait()

            def radd(r, c):
                def jadd(j, cc):
                    sl = pl.ds(j * _LANES, _LANES)
                    rows[p][r, sl] = rows[p][r, sl] + posb[p][r, sl]
                    return cc
                lax.fori_loop(0, NV, jadd, 0, unroll=8)
                return c
            lax.fori_loop(0, CH, radd, 0)

            pltpu.async_copy(rows[p], out_hbm.at[oidx[p]], so[p])

        def pair(t, carry):
            process(0, 2 * t)
            process(1, 2 * t + 1)
            return carry

        lax.fori_loop(0, NCH // 2, pair, 0)

        pltpu.make_async_copy(
            rows[0], out_hbm.at[oidx[0]], so[0]).wait()
        pltpu.make_async_copy(
            rows[1], out_hbm.at[oidx[1]], so[1]).wait()

    out_t_flat = pl.kernel(
        body,
        out_type=jax.ShapeDtypeStruct(((R + 1) * B, D), jnp.float32),
        mesh=mesh,
        scratch_types=[
            pltpu.VMEM((RW,), jnp.int32),
            pltpu.VMEM((CH,), jnp.int32),
            pltpu.VMEM((CH,), jnp.int32),
            pltpu.VMEM((CH,), jnp.int32),
            pltpu.VMEM((CH,), jnp.int32),
            pltpu.VMEM((CH,), jnp.int32),
            pltpu.VMEM((CH,), jnp.int32),
            pltpu.VMEM((CH, D), jnp.float32),
            pltpu.VMEM((CH, D), jnp.float32),
            pltpu.VMEM((CH, D), jnp.float32),
            pltpu.VMEM((CH, D), jnp.float32),
            pltpu.VMEM((B, D), jnp.float32),
            pltpu.VMEM((1, D), jnp.float32),
            pltpu.SemaphoreType.DMA,
            pltpu.SemaphoreType.DMA,
            pltpu.SemaphoreType.DMA,
            pltpu.SemaphoreType.DMA,
            pltpu.SemaphoreType.DMA,
            pltpu.SemaphoreType.DMA,
            pltpu.SemaphoreType.DMA,
        ],
    )(data_flat, idx_flat, pos, global_token)
    return out_t_flat


def _ridx_tc_kernel(idx_ref, out_ref):
    b = pl.program_id(0)
    row = idx_ref[b, :]
    out_ref[...] = jnp.broadcast_to(row[None, :, None], out_ref.shape)


@functools.partial(jax.jit, static_argnames=("D",))
def _ridx_run(remain_idx, *, D):
    B, R = remain_idx.shape
    return pl.pallas_call(
        _ridx_tc_kernel,
        grid=(B,),
        in_specs=[pl.BlockSpec((B, R), lambda b: (0, 0))],
        out_specs=pl.BlockSpec((1, R, D), lambda b: (b, 0, 0)),
        out_shape=jax.ShapeDtypeStruct((B, R, D), jnp.int32),
    )(remain_idx)


def kernel(data, remain_idx, global_token):
    B, S, D = data.shape
    R = remain_idx.shape[1]
    pos = jnp.asarray(_positional_encoding_np(D)[: S + 1, :])
    out_t_flat = _run(
        data.reshape(B * S, D),
        remain_idx.reshape(B * R),
        pos,
        global_token,
        B=B, S=S, R=R, D=D,
    )
    ridx = _ridx_run(remain_idx, D=D)
    out = out_t_flat.reshape(R + 1, B, D).transpose(1, 0, 2)
    return (out, ridx)


# speedup vs baseline: 5.7249x; 1.0462x over previous
import functools

import jax
import jax.numpy as jnp
import numpy as np
from jax import lax
from jax.experimental import pallas as pl
from jax.experimental.pallas import tpu as pltpu
from jax.experimental.pallas import tpu_sc as plsc

_LANES = 16


def _positional_encoding_np(d_model, seq_len=1000):
    position = np.arange(seq_len, dtype=np.float32).reshape(-1, 1)
    i = np.arange(d_model) // 2
    exp_term = (2.0 * i.astype(np.float32) / d_model).astype(np.float32)
    div_term = np.power(np.float32(10000.0), exp_term).reshape(1, -1)
    pe = (position / div_term).astype(np.float32)
    pe[:, 0::2] = np.sin(pe[:, 0::2])
    pe[:, 1::2] = np.cos(pe[:, 1::2])
    return pe


@functools.partial(jax.jit, static_argnames=("B", "S", "R", "D"))
def _run(data_flat, idx_flat, pos, global_token, *, B, S, R, D):
    info = plsc.get_sparse_core_info()
    NC, NS = info.num_cores, info.num_subcores
    NW = NC * NS
    N = B * R
    assert N % NW == 0
    RW = N // NW
    assert R % RW == 0
    CH = _LANES
    NCH = RW // CH
    NB = 3
    assert NCH >= NB and D % _LANES == 0
    assert B % 8 == 0
    NV = D // _LANES

    mesh = plsc.VectorSubcoreMesh(core_axis_name="c", subcore_axis_name="s")

    def body(data_hbm, idx_hbm, pos_hbm, gt_hbm, out_hbm,
             idx_v,
             didx0, didx1, didx2, pidx0, pidx1, pidx2,
             oidx0, oidx1, oidx2,
             rows0, rows1, rows2, pos0, pos1, pos2,
             ga_v, gb_v,
             sd0, sd1, sd2, sp0, sp1, sp2, so0, so1, so2, sg):
        didx = (didx0, didx1, didx2)
        pidx = (pidx0, pidx1, pidx2)
        oidx = (oidx0, oidx1, oidx2)
        rows = (rows0, rows1, rows2)
        posb = (pos0, pos1, pos2)
        sd = (sd0, sd1, sd2)
        sp = (sp0, sp1, sp2)
        so = (so0, so1, so2)

        cid = lax.axis_index("c")
        sid = lax.axis_index("s")
        wid = sid * NC + cid
        base = wid * RW
        b = base // R
        r0 = base - b * R

        pltpu.sync_copy(idx_hbm.at[pl.ds(base, RW)], idx_v)

        def set_indices(p, ci):
            off = ci * CH
            v = idx_v[pl.ds(off, CH)]
            didx[p][...] = v + b * S
            pidx[p][...] = v + 1
            oidx[p][...] = (lax.iota(jnp.int32, _LANES)
                            + (r0 + off + 1)) * B + b

        def issue_gathers(p):
            pltpu.async_copy(data_hbm.at[didx[p]], rows[p], sd[p])
            pltpu.async_copy(pos_hbm.at[pidx[p]], posb[p], sp[p])

        @pl.when(wid == 0)
        def _():
            pltpu.sync_copy(pos_hbm.at[pl.ds(0, 1)], gb_v)
            pltpu.sync_copy(gt_hbm, ga_v.at[pl.ds(0, 1)])

            def gadd(j, carry):
                sl = pl.ds(j * _LANES, _LANES)
                ga_v[0, sl] = ga_v[0, sl] + gb_v[0, sl]
                return carry

            lax.fori_loop(0, NV, gadd, 0, unroll=8)

            def gdup(r, carry):
                def jdup(j, cc):
                    sl = pl.ds(j * _LANES, _LANES)
                    ga_v[r, sl] = ga_v[0, sl]
                    return cc
                lax.fori_loop(0, NV, jdup, 0, unroll=8)
                return carry

            lax.fori_loop(1, B, gdup, 0)
            pltpu.async_copy(ga_v, out_hbm.at[pl.ds(0, B)], sg)

        set_indices(0, 0)
        issue_gathers(0)
        set_indices(1, 1)
        issue_gathers(1)

        for ci in range(NCH):
            p = ci % NB
            pltpu.make_async_copy(data_hbm.at[didx[p]], rows[p], sd[p]).wait()
            pltpu.make_async_copy(pos_hbm.at[pidx[p]], posb[p], sp[p]).wait()

            def radd(r, c, p=p):
                def jadd(j, cc):
                    sl = pl.ds(j * _LANES, _LANES)
                    rows[p][r, sl] = rows[p][r, sl] + posb[p][r, sl]
                    return cc
                lax.fori_loop(0, NV, jadd, 0, unroll=8)
                return c
            lax.fori_loop(0, CH, radd, 0)

            pltpu.async_copy(rows[p], out_hbm.at[oidx[p]], so[p])

            nxt = ci + NB - 1
            if nxt < NCH:
                q = nxt % NB
                if nxt >= NB:
                    pltpu.make_async_copy(
                        rows[q], out_hbm.at[oidx[q]], so[q]).wait()
                set_indices(q, nxt)
                issue_gathers(q)

        for m in range(NCH - NB, NCH):
            pltpu.make_async_copy(
                rows[m % NB], out_hbm.at[oidx[m % NB]], so[m % NB]).wait()

        @pl.when(wid == 0)
        def _():
            pltpu.make_async_copy(ga_v, out_hbm.at[pl.ds(0, B)], sg).wait()

    out_t_flat = pl.kernel(
        body,
        out_type=jax.ShapeDtypeStruct(((R + 1) * B, D), jnp.float32),
        mesh=mesh,
        scratch_types=[
            pltpu.VMEM((RW,), jnp.int32),
            pltpu.VMEM((CH,), jnp.int32),
            pltpu.VMEM((CH,), jnp.int32),
            pltpu.VMEM((CH,), jnp.int32),
            pltpu.VMEM((CH,), jnp.int32),
            pltpu.VMEM((CH,), jnp.int32),
            pltpu.VMEM((CH,), jnp.int32),
            pltpu.VMEM((CH,), jnp.int32),
            pltpu.VMEM((CH,), jnp.int32),
            pltpu.VMEM((CH,), jnp.int32),
            pltpu.VMEM((CH, D), jnp.float32),
            pltpu.VMEM((CH, D), jnp.float32),
            pltpu.VMEM((CH, D), jnp.float32),
            pltpu.VMEM((CH, D), jnp.float32),
            pltpu.VMEM((CH, D), jnp.float32),
            pltpu.VMEM((CH, D), jnp.float32),
            pltpu.VMEM((B, D), jnp.float32),
            pltpu.VMEM((1, D), jnp.float32),
            pltpu.SemaphoreType.DMA,
            pltpu.SemaphoreType.DMA,
            pltpu.SemaphoreType.DMA,
            pltpu.SemaphoreType.DMA,
            pltpu.SemaphoreType.DMA,
            pltpu.SemaphoreType.DMA,
            pltpu.SemaphoreType.DMA,
            pltpu.SemaphoreType.DMA,
            pltpu.SemaphoreType.DMA,
            pltpu.SemaphoreType.DMA,
        ],
    )(data_flat, idx_flat, pos, global_token)
    return out_t_flat


def _ridx_tc_kernel(idx_ref, out_ref):
    b = pl.program_id(0)
    row = idx_ref[b, :]
    out_ref[...] = jnp.broadcast_to(row[None, :, None], out_ref.shape)


@functools.partial(jax.jit, static_argnames=("D",))
def _ridx_run(remain_idx, *, D):
    B, R = remain_idx.shape
    return pl.pallas_call(
        _ridx_tc_kernel,
        grid=(B,),
        in_specs=[pl.BlockSpec((B, R), lambda b: (0, 0))],
        out_specs=pl.BlockSpec((1, R, D), lambda b: (b, 0, 0)),
        out_shape=jax.ShapeDtypeStruct((B, R, D), jnp.int32),
    )(remain_idx)


def kernel(data, remain_idx, global_token):
    B, S, D = data.shape
    R = remain_idx.shape[1]
    pos = jnp.asarray(_positional_encoding_np(D)[: S + 1, :])
    out_t_flat = _run(
        data.reshape(B * S, D),
        remain_idx.reshape(B * R),
        pos,
        global_token,
        B=B, S=S, R=R, D=D,
    )
    ridx = _ridx_run(remain_idx, D=D)
    out = out_t_flat.reshape(R + 1, B, D).transpose(1, 0, 2)
    return (out, ridx)


# speedup vs baseline: 8.6399x; 1.5092x over previous
import functools

import jax
import jax.numpy as jnp
import numpy as np
from jax import lax
from jax.experimental import pallas as pl
from jax.experimental.pallas import tpu as pltpu
from jax.experimental.pallas import tpu_sc as plsc

_LANES = 16


def _positional_encoding_np(d_model, seq_len=1000):
    position = np.arange(seq_len, dtype=np.float32).reshape(-1, 1)
    i = np.arange(d_model) // 2
    exp_term = (2.0 * i.astype(np.float32) / d_model).astype(np.float32)
    div_term = np.power(np.float32(10000.0), exp_term).reshape(1, -1)
    pe = (position / div_term).astype(np.float32)
    pe[:, 0::2] = np.sin(pe[:, 0::2])
    pe[:, 1::2] = np.cos(pe[:, 1::2])
    return pe


@functools.partial(jax.jit, static_argnames=("B", "S", "R", "D"))
def _run(data_flat, idx_flat, pos, global_token, *, B, S, R, D):
    info = plsc.get_sparse_core_info()
    NC, NS = info.num_cores, info.num_subcores
    NW = NC * NS
    N = B * R
    assert N % NW == 0
    RW = N // NW
    assert R % RW == 0
    CH = _LANES
    NCH = RW // CH
    NB = 3
    assert NCH >= NB and D % _LANES == 0
    assert B % 8 == 0
    NV = D // _LANES
    NV_SHIFT = NV.bit_length() - 1
    assert (1 << NV_SHIFT) == NV

    mesh = plsc.VectorSubcoreMesh(core_axis_name="c", subcore_axis_name="s")

    def body(data_hbm, idx_hbm, pos_hbm, gt_hbm, out_hbm,
             idx_v,
             didx0, didx1, didx2, pidx0, pidx1, pidx2,
             oidx0, oidx1, oidx2,
             rows0, rows1, rows2, pos0, pos1, pos2,
             ga_v, gb_v,
             sd0, sd1, sd2, sp0, sp1, sp2, so0, so1, so2, sg):
        didx = (didx0, didx1, didx2)
        pidx = (pidx0, pidx1, pidx2)
        oidx = (oidx0, oidx1, oidx2)
        rows = (rows0, rows1, rows2)
        posb = (pos0, pos1, pos2)
        sd = (sd0, sd1, sd2)
        sp = (sp0, sp1, sp2)
        so = (so0, so1, so2)

        cid = lax.axis_index("c")
        sid = lax.axis_index("s")
        wid = sid * NC + cid
        base = wid * RW
        b = base // R
        r0 = base - b * R

        pltpu.sync_copy(idx_hbm.at[pl.ds(base, RW)], idx_v)

        def set_indices(p, ci):
            off = ci * CH
            v = idx_v[pl.ds(off, CH)]
            didx[p][...] = v + b * S
            pidx[p][...] = v + 1
            oidx[p][...] = (lax.iota(jnp.int32, _LANES)
                            + (r0 + off + 1)) * B + b

        def issue_gathers(p):
            pltpu.async_copy(data_hbm.at[didx[p]], rows[p], sd[p])
            pltpu.async_copy(pos_hbm.at[pidx[p]], posb[p], sp[p])

        @pl.when(wid == 0)
        def _():
            pltpu.sync_copy(pos_hbm.at[pl.ds(0, 1)], gb_v)
            pltpu.sync_copy(gt_hbm, ga_v.at[pl.ds(0, 1)])

            @plsc.parallel_loop(0, NV, unroll=8)
            def _(j):
                sl = pl.ds(j * _LANES, _LANES)
                ga_v[0, sl] = ga_v[0, sl] + gb_v[0, sl]

            @plsc.parallel_loop(0, (B - 1) * NV, unroll=8)
            def _(i):
                r = 1 + (i >> NV_SHIFT)
                sl = pl.ds((i & (NV - 1)) * _LANES, _LANES)
                ga_v[r, sl] = ga_v[0, sl]
            pltpu.async_copy(ga_v, out_hbm.at[pl.ds(0, B)], sg)

        set_indices(0, 0)
        issue_gathers(0)
        set_indices(1, 1)
        issue_gathers(1)

        for ci in range(NCH):
            p = ci % NB
            pltpu.make_async_copy(data_hbm.at[didx[p]], rows[p], sd[p]).wait()
            pltpu.make_async_copy(pos_hbm.at[pidx[p]], posb[p], sp[p]).wait()

            @plsc.parallel_loop(0, CH * NV, unroll=8)
            def _(i, p=p):
                r = i >> NV_SHIFT
                sl = pl.ds((i & (NV - 1)) * _LANES, _LANES)
                rows[p][r, sl] = rows[p][r, sl] + posb[p][r, sl]

            pltpu.async_copy(rows[p], out_hbm.at[oidx[p]], so[p])

            nxt = ci + NB - 1
            if nxt < NCH:
                q = nxt % NB
                if nxt >= NB:
                    pltpu.make_async_copy(
                        rows[q], out_hbm.at[oidx[q]], so[q]).wait()
                set_indices(q, nxt)
                issue_gathers(q)

        for m in range(NCH - NB, NCH):
            pltpu.make_async_copy(
                rows[m % NB], out_hbm.at[oidx[m % NB]], so[m % NB]).wait()

        @pl.when(wid == 0)
        def _():
            pltpu.make_async_copy(ga_v, out_hbm.at[pl.ds(0, B)], sg).wait()

    out_t_flat = pl.kernel(
        body,
        out_type=jax.ShapeDtypeStruct(((R + 1) * B, D), jnp.float32),
        mesh=mesh,
        scratch_types=[
            pltpu.VMEM((RW,), jnp.int32),
            pltpu.VMEM((CH,), jnp.int32),
            pltpu.VMEM((CH,), jnp.int32),
            pltpu.VMEM((CH,), jnp.int32),
            pltpu.VMEM((CH,), jnp.int32),
            pltpu.VMEM((CH,), jnp.int32),
            pltpu.VMEM((CH,), jnp.int32),
            pltpu.VMEM((CH,), jnp.int32),
            pltpu.VMEM((CH,), jnp.int32),
            pltpu.VMEM((CH,), jnp.int32),
            pltpu.VMEM((CH, D), jnp.float32),
            pltpu.VMEM((CH, D), jnp.float32),
            pltpu.VMEM((CH, D), jnp.float32),
            pltpu.VMEM((CH, D), jnp.float32),
            pltpu.VMEM((CH, D), jnp.float32),
            pltpu.VMEM((CH, D), jnp.float32),
            pltpu.VMEM((B, D), jnp.float32),
            pltpu.VMEM((1, D), jnp.float32),
            pltpu.SemaphoreType.DMA,
            pltpu.SemaphoreType.DMA,
            pltpu.SemaphoreType.DMA,
            pltpu.SemaphoreType.DMA,
            pltpu.SemaphoreType.DMA,
            pltpu.SemaphoreType.DMA,
            pltpu.SemaphoreType.DMA,
            pltpu.SemaphoreType.DMA,
            pltpu.SemaphoreType.DMA,
            pltpu.SemaphoreType.DMA,
        ],
    )(data_flat, idx_flat, pos, global_token)
    return out_t_flat


def _ridx_tc_kernel(idx_ref, out_ref):
    b = pl.program_id(0)
    row = idx_ref[b, :]
    out_ref[...] = jnp.broadcast_to(row[None, :, None], out_ref.shape)


@functools.partial(jax.jit, static_argnames=("D",))
def _ridx_run(remain_idx, *, D):
    B, R = remain_idx.shape
    return pl.pallas_call(
        _ridx_tc_kernel,
        grid=(B,),
        in_specs=[pl.BlockSpec((B, R), lambda b: (0, 0))],
        out_specs=pl.BlockSpec((1, R, D), lambda b: (b, 0, 0)),
        out_shape=jax.ShapeDtypeStruct((B, R, D), jnp.int32),
    )(remain_idx)


def kernel(data, remain_idx, global_token):
    B, S, D = data.shape
    R = remain_idx.shape[1]
    pos = jnp.asarray(_positional_encoding_np(D)[: S + 1, :])
    out_t_flat = _run(
        data.reshape(B * S, D),
        remain_idx.reshape(B * R),
        pos,
        global_token,
        B=B, S=S, R=R, D=D,
    )
    ridx = _ridx_run(remain_idx, D=D)
    out = out_t_flat.reshape(R + 1, B, D).transpose(1, 0, 2)
    return (out, ridx)
